# Initial kernel scaffold; baseline (speedup 1.0000x reference)
import jax
import jax.numpy as jnp
from jax.experimental import pallas as pl


def kernel(x, table, gamma, beta, W1, b1, W2, b2, W3, b3):
    raise NotImplementedError("write your pallas kernel here")


# speedup vs baseline: 1.3308x; 1.3308x over previous
import functools

import jax
import jax.numpy as jnp
from jax import lax
from jax.experimental import pallas as pl
from jax.experimental.pallas import tpu as pltpu
from jax.experimental.pallas import tpu_sc as plsc

B = 16384
F = 26
D = 16
V = 1000000
EMB = F * D
EPS = 1e-5

NC = 2
NS = 16
NW = NC * NS
N = B * F
PER_W = N // NW
CHUNK = 1664
NI = PER_W // CHUNK


def _make_gather():
    mesh = plsc.VectorSubcoreMesh(core_axis_name="c", subcore_axis_name="s")

    @functools.partial(
        pl.kernel,
        mesh=mesh,
        out_type=jax.ShapeDtypeStruct((N, D), jnp.float32),
        scratch_types=[
            pltpu.VMEM((CHUNK,), jnp.int32),
            pltpu.VMEM((CHUNK, D), jnp.float32),
            pltpu.SemaphoreType.DMA,
        ],
        compiler_params=pltpu.CompilerParams(use_tc_tiling_on_sc=False),
    )
    def gather(table_hbm, idx_hbm, out_hbm, idx_v, rows_v, sem):
        wid = lax.axis_index("s") * NC + lax.axis_index("c")
        base = wid * PER_W

        def body(i, carry):
            off = base + i * CHUNK
            pltpu.sync_copy(idx_hbm.at[pl.ds(off, CHUNK)], idx_v)
            pltpu.async_copy(table_hbm.at[idx_v], rows_v, sem).wait()
            pltpu.sync_copy(rows_v, out_hbm.at[pl.ds(off, CHUNK)])
            return carry

        lax.fori_loop(0, NI, body, 0)

    return gather


_gather = _make_gather()


STATS_TB = 2048
STATS_NB = B // STATS_TB


def _stats_kernel(h_ref, gamma_ref, beta_ref, scale_ref, shift_ref,
                  sum_ref, sumsq_ref):
    i = pl.program_id(0)
    blk = h_ref[...]
    s = jnp.sum(blk, axis=0, keepdims=True)
    s2 = jnp.sum(blk * blk, axis=0, keepdims=True)

    @pl.when(i == 0)
    def _init():
        sum_ref[...] = s
        sumsq_ref[...] = s2

    @pl.when(i > 0)
    def _acc():
        sum_ref[...] += s
        sumsq_ref[...] += s2

    @pl.when(i == STATS_NB - 1)
    def _finish():
        mean = sum_ref[...] * (1.0 / B)
        var = sumsq_ref[...] * (1.0 / B) - mean * mean
        rstd = lax.rsqrt(var + EPS)
        scl = gamma_ref[...] * rstd
        scale_ref[...] = scl
        shift_ref[...] = beta_ref[...] - mean * scl


def _stats(h, gamma, beta):
    return pl.pallas_call(
        _stats_kernel,
        grid=(STATS_NB,),
        in_specs=[
            pl.BlockSpec((STATS_TB, EMB), lambda i: (i, 0)),
            pl.BlockSpec((1, EMB), lambda i: (0, 0)),
            pl.BlockSpec((1, EMB), lambda i: (0, 0)),
        ],
        out_specs=[
            pl.BlockSpec((1, EMB), lambda i: (0, 0)),
            pl.BlockSpec((1, EMB), lambda i: (0, 0)),
        ],
        out_shape=[
            jax.ShapeDtypeStruct((1, EMB), jnp.float32),
            jax.ShapeDtypeStruct((1, EMB), jnp.float32),
        ],
        scratch_shapes=[
            pltpu.VMEM((1, EMB), jnp.float32),
            pltpu.VMEM((1, EMB), jnp.float32),
        ],
    )(h, gamma.reshape(1, EMB), beta.reshape(1, EMB))


MLP_TB = 1024
MLP_NB = B // MLP_TB
H1, H2, H3 = 256, 128, 64


def _mlp_kernel(h_ref, scale_ref, shift_ref, W1_ref, b1_ref, W2_ref, b2_ref,
                W3_ref, b3_ref, out_ref):
    h = h_ref[...] * scale_ref[...] + shift_ref[...]
    a = jnp.dot(h, W1_ref[...], preferred_element_type=jnp.float32)
    a = jnp.maximum(a + b1_ref[...], 0.0)
    a = jnp.dot(a, W2_ref[...], preferred_element_type=jnp.float32)
    a = jnp.maximum(a + b2_ref[...], 0.0)
    a = jnp.dot(a, W3_ref[...], preferred_element_type=jnp.float32)
    out_ref[...] = a + b3_ref[...]


def _mlp(h, scale, shift, W1, b1, W2, b2, W3, b3):
    return pl.pallas_call(
        _mlp_kernel,
        grid=(MLP_NB,),
        in_specs=[
            pl.BlockSpec((MLP_TB, EMB), lambda i: (i, 0)),
            pl.BlockSpec((1, EMB), lambda i: (0, 0)),
            pl.BlockSpec((1, EMB), lambda i: (0, 0)),
            pl.BlockSpec((EMB, H1), lambda i: (0, 0)),
            pl.BlockSpec((1, H1), lambda i: (0, 0)),
            pl.BlockSpec((H1, H2), lambda i: (0, 0)),
            pl.BlockSpec((1, H2), lambda i: (0, 0)),
            pl.BlockSpec((H2, H3), lambda i: (0, 0)),
            pl.BlockSpec((1, H3), lambda i: (0, 0)),
        ],
        out_specs=pl.BlockSpec((MLP_TB, H3), lambda i: (i, 0)),
        out_shape=jax.ShapeDtypeStruct((B, H3), jnp.float32),
    )(h, scale, shift, W1, b1.reshape(1, H1), W2, b2.reshape(1, H2),
      W3, b3.reshape(1, H3))


@jax.jit
def kernel(x, table, gamma, beta, W1, b1, W2, b2, W3, b3):
    flat_idx = x.reshape(N)
    h = _gather(table, flat_idx).reshape(B, EMB)
    scale, shift = _stats(h, gamma, beta)
    return _mlp(h, scale, shift, W1, b1, W2, b2, W3, b3)
